# baseline (speedup 1.0000x reference)
import functools

import jax
import jax.numpy as jnp
from jax import lax
from jax.experimental import pallas as pl
from jax.experimental.pallas import tpu as pltpu
from jax.experimental.pallas import tpu_sc as plsc

N, E = 10000, 640000
H = 16
IN = 19

NC, NS = 2, 16
NW = NC * NS
CHUNK = 128
CPW = 160
EPW = CPW * CHUNK
E_PAD = NW * EPW
FIRE = 8
NPT = N // NS

_f32 = jnp.float32


def _leaky(x):
    return jnp.where(x > 0, x, 0.01 * x)


def _ef_stats_body(ef_ref, out_ref):
    i = pl.program_id(0)
    blk = jnp.nan_to_num(ef_ref[...])

    @pl.when(i == 0)
    def _():
        out_ref[...] = jnp.zeros_like(out_ref)

    out_ref[...] += jnp.stack([jnp.sum(blk, axis=0), jnp.sum(blk * blk, axis=0)])


def _edge_mlp_body(ef_ref, st_ref, w1_ref, b1_ref, w2_ref, b2_ref,
                   wa1_ref, ba1_ref, wa2_ref, ba2_ref,
                   efn_ref, efnn_ref, a2_ref):
    st = st_ref[...]
    s1 = st[0]
    s2 = st[1]
    mean = s1 / E
    var = jnp.maximum(s2 / E - mean * mean, 0.0)
    rstd = 1.0 / (jnp.sqrt(var) + 1e-4)
    efn = (jnp.nan_to_num(ef_ref[...]) - mean) * rstd
    efn_ref[...] = efn
    h = _leaky(jnp.dot(efn, w1_ref[...], preferred_element_type=_f32) + b1_ref[...])
    efnn = _leaky(jnp.dot(h, w2_ref[...], preferred_element_type=_f32) + b2_ref[...])
    efnn_ref[...] = efnn
    g = _leaky(jnp.dot(efnn, wa1_ref[...], preferred_element_type=_f32) + ba1_ref[...])
    a2_ref[...] = jax.nn.sigmoid(
        jnp.dot(g, wa2_ref[...], preferred_element_type=_f32) + ba2_ref[...])


def _node_mlp_body(x_ref, w1_ref, b1_ref, w2_ref, b2_ref, node_ref):
    x = jnp.nan_to_num(x_ref[...])
    mean = jnp.mean(x, axis=0)
    xc = x - mean
    std = jnp.sqrt(jnp.mean(xc * xc, axis=0))
    xn = xc / (std + 1e-4)
    h = _leaky(jnp.dot(xn, w1_ref[...], preferred_element_type=_f32) + b1_ref[...])
    node_ref[...] = _leaky(jnp.dot(h, w2_ref[...], preferred_element_type=_f32)
                           + b2_ref[...])


def _msg_mlp_body(hs_ref, hd_ref, a2_ref, w1p_ref, b1p_ref, w2p_ref, b2p_ref,
                  md_ref, mr_ref):
    x = jnp.concatenate([hs_ref[...], hd_ref[...]], axis=1)
    pre = _leaky(jnp.dot(x, w1p_ref[...], preferred_element_type=_f32)
                 + b1p_ref[...])
    out = _leaky(jnp.dot(pre, w2p_ref[...], preferred_element_type=_f32)
                 + b2p_ref[...])
    a2 = a2_ref[...]
    md_ref[...] = out[:, :H] * a2[:, 0:1]
    mr_ref[...] = out[:, H:] * a2[:, 1:2]


def _den_reduce_body(dparts_ref, den_ref):
    den_ref[...] = 1.0 + jnp.sum(dparts_ref[...], axis=0)


def _node_update_body(node_ref, parts_ref, den_ref, w1a_ref, b1_ref, w2_ref, b2_ref,
                      out_ref):
    node = node_ref[...]
    h = _leaky(jnp.dot(node, w1a_ref[...], preferred_element_type=_f32) + b1_ref[...])
    m_self = _leaky(jnp.dot(h, w2_ref[...], preferred_element_type=_f32) + b2_ref[...])
    parts = parts_ref[...]
    acc = m_self + parts[0] + parts[1]
    out_ref[...] = acc / den_ref[...]


def _edge_out_body(hs_ref, hd_ref, efnn_ref, efn_ref,
                   w1_ref, b1_ref, w2_ref, b2_ref, pred_ref):
    emb = jnp.concatenate([hs_ref[...], hd_ref[...], efnn_ref[...],
                           efn_ref[...]], axis=1)
    h = _leaky(jnp.dot(emb, w1_ref[...], preferred_element_type=_f32)
               + b1_ref[...])
    out = jnp.dot(h, w2_ref[...], preferred_element_type=_f32) + b2_ref[...]
    pred_ref[...] = jax.nn.sigmoid(out[:, 0])


@functools.lru_cache(maxsize=1)
def _sc_kernels():
    mesh = plsc.VectorSubcoreMesh(core_axis_name="c", subcore_axis_name="s",
                                  num_cores=NC, num_subcores=NS)
    sc_params = pltpu.CompilerParams(use_tc_tiling_on_sc=False,
                                     needs_layout_passes=False)

    def _gather_stripe(node_h, idx_v, out_h, base, rows, sem):
        def batch(j, _):
            cps = []
            for k in range(FIRE):
                cps.append(pltpu.async_copy(
                    node_h.at[idx_v.at[j * FIRE + k]],
                    rows.at[pl.ds(k * CHUNK, CHUNK)], sem))
            for cp in cps:
                cp.wait()
            pltpu.sync_copy(rows, out_h.at[pl.ds(base + j * (FIRE * CHUNK),
                                                 FIRE * CHUNK)])
            return 0

        lax.fori_loop(0, CPW // FIRE, batch, 0, unroll=False)

    @functools.partial(
        pl.kernel, mesh=mesh, compiler_params=sc_params,
        out_type=(jax.ShapeDtypeStruct((E_PAD, H), _f32),
                  jax.ShapeDtypeStruct((E_PAD, H), _f32)),
        scratch_types=[
            pltpu.VMEM((CPW, CHUNK), jnp.int32),
            pltpu.VMEM((CPW, CHUNK), jnp.int32),
            pltpu.VMEM((FIRE * CHUNK, H), _f32),
            pltpu.SemaphoreType.DMA,
        ],
    )
    def gather2(node_h, src_h, dst_h, hs_h, hd_h, srcv, dstv, rows, sem):
        wid = lax.axis_index("s") * NC + lax.axis_index("c")
        base = wid * EPW
        pltpu.sync_copy(src_h.at[pl.ds(wid * CPW, CPW)], srcv)
        pltpu.sync_copy(dst_h.at[pl.ds(wid * CPW, CPW)], dstv)
        _gather_stripe(node_h, srcv, hs_h, base, rows, sem)
        _gather_stripe(node_h, dstv, hd_h, base, rows, sem)

    @functools.partial(
        pl.kernel, mesh=mesh, compiler_params=sc_params,
        out_type=jax.ShapeDtypeStruct((NC, N, H), _f32),
        scratch_types=[
            pltpu.VMEM((CPW, CHUNK), jnp.int32),
            pltpu.VMEM((CPW, CHUNK), jnp.int32),
            pltpu.VMEM((CHUNK, H), _f32),
            pltpu.VMEM_SHARED((N, H), _f32),
        ],
    )
    def scatter2(md_h, mr_h, src_h, dst_h, zero_h, parts_h, srcv, dstv, mrow, acc):
        cid = lax.axis_index("c")
        sid = lax.axis_index("s")
        wid = sid * NC + cid
        base = wid * EPW

        @pl.when(sid == 0)
        def _():
            pltpu.sync_copy(zero_h, acc)

        plsc.subcore_barrier()
        pltpu.sync_copy(src_h.at[pl.ds(wid * CPW, CPW)], srcv)
        pltpu.sync_copy(dst_h.at[pl.ds(wid * CPW, CPW)], dstv)

        def step_d(j, _):
            pltpu.sync_copy(md_h.at[pl.ds(base + j * CHUNK, CHUNK)], mrow)
            pltpu.sync_copy(mrow, acc.at[srcv.at[j]], add=True)
            return 0

        def step_r(j, _):
            pltpu.sync_copy(mr_h.at[pl.ds(base + j * CHUNK, CHUNK)], mrow)
            pltpu.sync_copy(mrow, acc.at[dstv.at[j]], add=True)
            return 0

        lax.fori_loop(0, CPW, step_d, 0, unroll=False)
        lax.fori_loop(0, CPW, step_r, 0, unroll=False)
        plsc.subcore_barrier()
        pltpu.sync_copy(acc.at[pl.ds(sid * NPT, NPT)],
                        parts_h.at[cid, pl.ds(sid * NPT, NPT)])

    @functools.partial(
        pl.kernel, mesh=mesh, compiler_params=sc_params,
        out_type=jax.ShapeDtypeStruct((NW, N), _f32),
        scratch_types=[
            pltpu.VMEM((EPW,), jnp.int32),
            pltpu.VMEM((EPW,), jnp.int32),
            pltpu.VMEM((EPW,), _f32),
            pltpu.VMEM((EPW,), _f32),
            pltpu.VMEM((N,), _f32),
        ],
    )
    def den_scatter(src_h, dst_h, ad_h, ar_h, dparts_h, sv, dv, av, bv, acc):
        wid = lax.axis_index("s") * NC + lax.axis_index("c")
        base = wid * EPW

        def zero(i, _):
            acc[pl.ds(i * 16, 16)] = jnp.zeros((16,), _f32)
            return 0

        lax.fori_loop(0, N // 16, zero, 0, unroll=False)
        pltpu.sync_copy(src_h.at[pl.ds(base, EPW)], sv)
        pltpu.sync_copy(dst_h.at[pl.ds(base, EPW)], dv)
        pltpu.sync_copy(ad_h.at[pl.ds(base, EPW)], av)
        pltpu.sync_copy(ar_h.at[pl.ds(base, EPW)], bv)

        def step(i, _):
            plsc.addupdate_scatter(acc, [sv[pl.ds(i * 16, 16)]],
                                   av[pl.ds(i * 16, 16)])
            plsc.addupdate_scatter(acc, [dv[pl.ds(i * 16, 16)]],
                                   bv[pl.ds(i * 16, 16)])
            return 0

        lax.fori_loop(0, EPW // 16, step, 0, unroll=False)
        pltpu.sync_copy(acc, dparts_h.at[wid])

    return gather2, scatter2, den_scatter


def _tc_call(body, grid, in_specs, out_specs, out_shape, parallel=False):
    cp = (pltpu.CompilerParams(dimension_semantics=("parallel",))
          if parallel else None)
    return pl.pallas_call(body, grid=grid, in_specs=in_specs,
                          out_specs=out_specs, out_shape=out_shape,
                          compiler_params=cp)


def kernel(x, edge_index, edges_features, params):
    p = params
    gather2, scatter2, den_scatter = _sc_kernels()

    src = edge_index[0].astype(jnp.int32)
    dst = edge_index[1].astype(jnp.int32)
    pad = jnp.zeros((E_PAD - E,), jnp.int32)
    src_p = jnp.concatenate([src, pad]).reshape(NW * CPW, CHUNK)
    dst_p = jnp.concatenate([dst, pad]).reshape(NW * CPW, CHUNK)
    src_f = src_p.reshape(E_PAD)
    dst_f = dst_p.reshape(E_PAD)

    BLK_E = 5000
    stats = _tc_call(
        _ef_stats_body, (E // BLK_E,),
        [pl.BlockSpec((BLK_E, 16), lambda i: (i, 0))],
        pl.BlockSpec((2, 16), lambda i: (0, 0)),
        jax.ShapeDtypeStruct((2, 16), _f32),
    )(edges_features)

    wa1 = jnp.concatenate([p['ad_W1'], p['ar_W1']], axis=1)
    ba1 = jnp.concatenate([p['ad_b1'], p['ar_b1']]).reshape(1, 32)
    wa2 = jnp.zeros((32, 2), _f32)
    wa2 = wa2.at[:16, 0:1].set(p['ad_W2']).at[16:, 1:2].set(p['ar_W2'])
    ba2 = jnp.concatenate([p['ad_b2'], p['ar_b2']]).reshape(1, 2)

    cw = lambda s: pl.BlockSpec(s, lambda i: tuple(0 for _ in s))
    efn, ef_nn, a2 = _tc_call(
        _edge_mlp_body, (E // BLK_E,),
        [pl.BlockSpec((BLK_E, 16), lambda i: (i, 0)), cw((2, 16)),
         cw((16, 16)), cw((1, 16)), cw((16, 16)), cw((1, 16)),
         cw((16, 32)), cw((1, 32)), cw((32, 2)), cw((1, 2))],
        [pl.BlockSpec((BLK_E, 16), lambda i: (i, 0)),
         pl.BlockSpec((BLK_E, 16), lambda i: (i, 0)),
         pl.BlockSpec((BLK_E, 2), lambda i: (i, 0))],
        [jax.ShapeDtypeStruct((E, 16), _f32),
         jax.ShapeDtypeStruct((E, 16), _f32),
         jax.ShapeDtypeStruct((E, 2), _f32)],
        parallel=True,
    )(edges_features, stats, p['ein_W1'], p['ein_b1'].reshape(1, 16),
      p['ein_W2'], p['ein_b2'].reshape(1, 16), wa1, ba1, wa2, ba2)

    a2_p = jnp.concatenate([a2, jnp.zeros((E_PAD - E, 2), _f32)])
    ad_f = a2_p[:, 0]
    ar_f = a2_p[:, 1]

    node = _tc_call(
        _node_mlp_body, (1,),
        [cw((N, IN)), cw((IN, H)), cw((1, H)), cw((H, H)), cw((1, H))],
        cw((N, H)),
        jax.ShapeDtypeStruct((N, H), _f32),
    )(x, p['inp_W1'], p['inp_b1'].reshape(1, H),
      p['inp_W2'], p['inp_b2'].reshape(1, H))

    dparts = den_scatter(src_f, dst_f, ad_f, ar_f)
    den = _tc_call(
        _den_reduce_body, (1,),
        [cw((NW, N))], cw((N,)),
        jax.ShapeDtypeStruct((N,), _f32),
    )(dparts)
    den_col = den.reshape(N, 1)

    zero_nh = jnp.zeros((N, H), _f32)
    BLK_M = 4096
    for name in ('gc0', 'gc1'):
        w1 = p[name + '_W1']
        w1a, w1b = w1[:H], w1[H:]
        b1 = p[name + '_b1'].reshape(1, 2 * H)
        w2 = p[name + '_W2']
        b2 = p[name + '_b2'].reshape(1, H)

        wd = jnp.concatenate([w1a - w1b, w1b], axis=0)
        wr = jnp.concatenate([w1b, w1a - w1b], axis=0)
        w1p = jnp.concatenate([wd, wr], axis=1)
        b1p = jnp.concatenate([b1, b1], axis=1)
        w2p = jnp.zeros((4 * H, 2 * H), _f32)
        w2p = w2p.at[:2 * H, :H].set(w2).at[2 * H:, H:].set(w2)
        b2p = jnp.concatenate([b2, b2], axis=1)

        hs, hd = gather2(node, src_p, dst_p)
        md, mr = _tc_call(
            _msg_mlp_body, (E_PAD // BLK_M,),
            [pl.BlockSpec((BLK_M, H), lambda i: (i, 0)),
             pl.BlockSpec((BLK_M, H), lambda i: (i, 0)),
             pl.BlockSpec((BLK_M, 2), lambda i: (i, 0)),
             cw((2 * H, 4 * H)), cw((1, 4 * H)),
             cw((4 * H, 2 * H)), cw((1, 2 * H))],
            [pl.BlockSpec((BLK_M, H), lambda i: (i, 0)),
             pl.BlockSpec((BLK_M, H), lambda i: (i, 0))],
            [jax.ShapeDtypeStruct((E_PAD, H), _f32),
             jax.ShapeDtypeStruct((E_PAD, H), _f32)],
            parallel=True,
        )(hs, hd, a2_p, w1p, b1p, w2p, b2p)

        parts = scatter2(md, mr, src_p, dst_p, zero_nh)
        node = _tc_call(
            _node_update_body, (1,),
            [cw((N, H)), cw((NC, N, H)), cw((N, 1)),
             cw((H, 2 * H)), cw((1, 2 * H)), cw((2 * H, H)), cw((1, H))],
            cw((N, H)),
            jax.ShapeDtypeStruct((N, H), _f32),
        )(node, parts, den_col, w1a, b1, w2, b2)

    hs, hd = gather2(node, src_p, dst_p)
    BLK_O = 5120
    pred = _tc_call(
        _edge_out_body, (E // BLK_O,),
        [pl.BlockSpec((BLK_O, H), lambda i: (i, 0)),
         pl.BlockSpec((BLK_O, H), lambda i: (i, 0)),
         pl.BlockSpec((BLK_O, 16), lambda i: (i, 0)),
         pl.BlockSpec((BLK_O, 16), lambda i: (i, 0)),
         cw((2 * H + 32, H)), cw((1, H)), cw((H, 1)), cw((1, 1))],
        pl.BlockSpec((BLK_O,), lambda i: (i,)),
        jax.ShapeDtypeStruct((E,), _f32),
        parallel=True,
    )(hs, hd, ef_nn, efn, p['en_W1'],
      p['en_b1'].reshape(1, H), p['en_W2'], p['en_b2'].reshape(1, 1))
    return pred


# speedup vs baseline: 1.0633x; 1.0633x over previous
import functools

import jax
import jax.numpy as jnp
from jax import lax
from jax.experimental import pallas as pl
from jax.experimental.pallas import tpu as pltpu
from jax.experimental.pallas import tpu_sc as plsc

N, E = 10000, 640000
H = 16
IN = 19

NC, NS = 2, 16
NW = NC * NS
CHUNK = 128
CPW = 160
EPW = CPW * CHUNK
E_PAD = NW * EPW
FIRE = 8
NPT = N // NS
M8 = E_PAD // 8

_f32 = jnp.float32


def _leaky(x):
    return jnp.where(x > 0, x, 0.01 * x)


def _ef_stats_body(ef_ref, out_ref):
    i = pl.program_id(0)
    blk = jnp.nan_to_num(ef_ref[...])

    @pl.when(i == 0)
    def _():
        out_ref[...] = jnp.zeros_like(out_ref)

    out_ref[...] += jnp.stack([jnp.sum(blk, axis=0), jnp.sum(blk * blk, axis=0)])


def _edge_mlp_body(ef_ref, st_ref, w1_ref, b1_ref, w2_ref, b2_ref,
                   wa1_ref, ba1_ref, wa2_ref, ba2_ref,
                   efn_ref, efnn_ref, a2_ref, ad_ref, ar_ref):
    i = pl.program_id(0)
    valid = (i < E // 5120).astype(_f32)
    st = st_ref[...]
    s1 = st[0]
    s2 = st[1]
    mean = s1 / E
    var = jnp.maximum(s2 / E - mean * mean, 0.0)
    rstd = 1.0 / (jnp.sqrt(var) + 1e-4)
    efn = (jnp.nan_to_num(ef_ref[...]) - mean) * rstd * valid
    efn_ref[...] = efn
    h = _leaky(jnp.dot(efn, w1_ref[...], preferred_element_type=_f32) + b1_ref[...])
    efnn = _leaky(jnp.dot(h, w2_ref[...], preferred_element_type=_f32) + b2_ref[...])
    efnn_ref[...] = efnn
    g = _leaky(jnp.dot(efnn, wa1_ref[...], preferred_element_type=_f32) + ba1_ref[...])
    a2 = jax.nn.sigmoid(
        jnp.dot(g, wa2_ref[...], preferred_element_type=_f32) + ba2_ref[...]) * valid
    a2_ref[...] = a2
    ad_ref[...] = a2[:, 0]
    ar_ref[...] = a2[:, 1]


def _node_mlp_body(x_ref, w1_ref, b1_ref, w2_ref, b2_ref, node_ref):
    x = jnp.nan_to_num(x_ref[...])
    mean = jnp.mean(x, axis=0)
    xc = x - mean
    std = jnp.sqrt(jnp.mean(xc * xc, axis=0))
    xn = xc / (std + 1e-4)
    h = _leaky(jnp.dot(xn, w1_ref[...], preferred_element_type=_f32) + b1_ref[...])
    node_ref[...] = _leaky(jnp.dot(h, w2_ref[...], preferred_element_type=_f32)
                           + b2_ref[...])


def _msg_mlp_body(hs_ref, hd_ref, a2_ref, w1p_ref, b1p_ref, w2p_ref, b2p_ref,
                  md_ref, mr_ref):
    x = jnp.concatenate([hs_ref[...], hd_ref[...]], axis=1)
    pre = _leaky(jnp.dot(x, w1p_ref[...], preferred_element_type=_f32)
                 + b1p_ref[...])
    out = _leaky(jnp.dot(pre, w2p_ref[...], preferred_element_type=_f32)
                 + b2p_ref[...])
    a2 = a2_ref[...]
    md_ref[...] = out[:, :H] * a2[:, 0:1]
    mr_ref[...] = out[:, H:] * a2[:, 1:2]


def _den_reduce_body(dparts_ref, den_ref):
    den_ref[...] = 1.0 + jnp.sum(dparts_ref[...], axis=0)


def _node_update_body(node_ref, parts_ref, den_ref, w1a_ref, b1_ref, w2_ref, b2_ref,
                      out_ref):
    node = node_ref[...]
    h = _leaky(jnp.dot(node, w1a_ref[...], preferred_element_type=_f32) + b1_ref[...])
    m_self = _leaky(jnp.dot(h, w2_ref[...], preferred_element_type=_f32) + b2_ref[...])
    parts = parts_ref[...]
    acc = m_self + parts[0] + parts[1]
    out_ref[...] = acc / den_ref[...]


def _edge_out_body(hs_ref, hd_ref, efnn_ref, efn_ref,
                   w1_ref, b1_ref, w2_ref, b2_ref, pred_ref):
    emb = jnp.concatenate([hs_ref[...], hd_ref[...], efnn_ref[...],
                           efn_ref[...]], axis=1)
    h = _leaky(jnp.dot(emb, w1_ref[...], preferred_element_type=_f32)
               + b1_ref[...])
    out = jnp.dot(h, w2_ref[...], preferred_element_type=_f32) + b2_ref[...]
    pred_ref[...] = jax.nn.sigmoid(out[:, 0])


@functools.lru_cache(maxsize=1)
def _sc_kernels():
    mesh = plsc.VectorSubcoreMesh(core_axis_name="c", subcore_axis_name="s",
                                  num_cores=NC, num_subcores=NS)
    sc_params = pltpu.CompilerParams(use_tc_tiling_on_sc=False,
                                     needs_layout_passes=False)

    def _gather_stripe(node_h, idx_v, out_h, base, rows, sem):
        def batch(j, _):
            cps = []
            for k in range(FIRE):
                cps.append(pltpu.async_copy(
                    node_h.at[idx_v.at[j * FIRE + k]],
                    rows.at[pl.ds(k * CHUNK, CHUNK)], sem))
            for cp in cps:
                cp.wait()
            pltpu.sync_copy(rows, out_h.at[pl.ds(base + j * (FIRE * CHUNK),
                                                 FIRE * CHUNK)])
            return 0

        lax.fori_loop(0, CPW // FIRE, batch, 0, unroll=False)

    @functools.partial(
        pl.kernel, mesh=mesh, compiler_params=sc_params,
        out_type=(jax.ShapeDtypeStruct((E_PAD, H), _f32),
                  jax.ShapeDtypeStruct((E_PAD, H), _f32)),
        scratch_types=[
            pltpu.VMEM((CPW, CHUNK), jnp.int32),
            pltpu.VMEM((CPW, CHUNK), jnp.int32),
            pltpu.VMEM((FIRE * CHUNK, H), _f32),
            pltpu.SemaphoreType.DMA,
        ],
    )
    def gather2(node_h, src_h, dst_h, hs_h, hd_h, srcv, dstv, rows, sem):
        wid = lax.axis_index("s") * NC + lax.axis_index("c")
        base = wid * EPW
        pltpu.sync_copy(src_h.at[pl.ds(wid * CPW, CPW)], srcv)
        pltpu.sync_copy(dst_h.at[pl.ds(wid * CPW, CPW)], dstv)
        _gather_stripe(node_h, srcv, hs_h, base, rows, sem)
        _gather_stripe(node_h, dstv, hd_h, base, rows, sem)

    @functools.partial(
        pl.kernel, mesh=mesh, compiler_params=sc_params,
        out_type=jax.ShapeDtypeStruct((NC, N, H), _f32),
        scratch_types=[
            pltpu.VMEM((CPW, CHUNK), jnp.int32),
            pltpu.VMEM((CPW, CHUNK), jnp.int32),
            pltpu.VMEM((CHUNK, H), _f32),
            pltpu.VMEM_SHARED((N, H), _f32),
        ],
    )
    def scatter2(md_h, mr_h, src_h, dst_h, zero_h, parts_h, srcv, dstv, mrow, acc):
        cid = lax.axis_index("c")
        sid = lax.axis_index("s")
        wid = sid * NC + cid
        base = wid * EPW
        md2 = md_h
        mr2 = mr_h

        @pl.when(sid == 0)
        def _():
            pltpu.sync_copy(zero_h, acc)

        plsc.subcore_barrier()
        pltpu.sync_copy(src_h.at[pl.ds(wid * CPW, CPW)], srcv)
        pltpu.sync_copy(dst_h.at[pl.ds(wid * CPW, CPW)], dstv)

        def step_d(j, _):
            pltpu.sync_copy(md2.at[pl.ds(base + j * CHUNK, CHUNK)], mrow)
            pltpu.sync_copy(mrow, acc.at[srcv.at[j]], add=True)
            return 0

        def step_r(j, _):
            pltpu.sync_copy(mr2.at[pl.ds(base + j * CHUNK, CHUNK)], mrow)
            pltpu.sync_copy(mrow, acc.at[dstv.at[j]], add=True)
            return 0

        lax.fori_loop(0, CPW, step_d, 0, unroll=False)
        lax.fori_loop(0, CPW, step_r, 0, unroll=False)
        plsc.subcore_barrier()
        pltpu.sync_copy(acc.at[pl.ds(sid * NPT, NPT)],
                        parts_h.at[cid, pl.ds(sid * NPT, NPT)])

    @functools.partial(
        pl.kernel, mesh=mesh, compiler_params=sc_params,
        out_type=jax.ShapeDtypeStruct((NW, N), _f32),
        scratch_types=[
            pltpu.VMEM((EPW,), jnp.int32),
            pltpu.VMEM((EPW,), jnp.int32),
            pltpu.VMEM((EPW,), _f32),
            pltpu.VMEM((EPW,), _f32),
            pltpu.VMEM((N,), _f32),
        ],
    )
    def den_scatter(src_h, dst_h, ad_h, ar_h, dparts_h, sv, dv, av, bv, acc):
        wid = lax.axis_index("s") * NC + lax.axis_index("c")
        base = wid * EPW

        def zero(i, _):
            acc[pl.ds(i * 16, 16)] = jnp.zeros((16,), _f32)
            return 0

        lax.fori_loop(0, N // 16, zero, 0, unroll=False)
        pltpu.sync_copy(src_h.at[pl.ds(base, EPW)], sv)
        pltpu.sync_copy(dst_h.at[pl.ds(base, EPW)], dv)
        pltpu.sync_copy(ad_h.at[pl.ds(base, EPW)], av)
        pltpu.sync_copy(ar_h.at[pl.ds(base, EPW)], bv)

        def step(i, _):
            plsc.addupdate_scatter(acc, [sv[pl.ds(i * 16, 16)]],
                                   av[pl.ds(i * 16, 16)])
            plsc.addupdate_scatter(acc, [dv[pl.ds(i * 16, 16)]],
                                   bv[pl.ds(i * 16, 16)])
            return 0

        lax.fori_loop(0, EPW // 16, step, 0, unroll=False)
        pltpu.sync_copy(acc, dparts_h.at[wid])

    return gather2, scatter2, den_scatter


def _tc_call(body, grid, in_specs, out_specs, out_shape, parallel=False):
    cp = (pltpu.CompilerParams(dimension_semantics=("parallel",))
          if parallel else None)
    return pl.pallas_call(body, grid=grid, in_specs=in_specs,
                          out_specs=out_specs, out_shape=out_shape,
                          compiler_params=cp)


def kernel(x, edge_index, edges_features, params):
    p = params
    gather2, scatter2, den_scatter = _sc_kernels()

    src = edge_index[0].astype(jnp.int32)
    dst = edge_index[1].astype(jnp.int32)
    pad = jnp.zeros((E_PAD - E,), jnp.int32)
    src_p = jnp.concatenate([src, pad]).reshape(NW * CPW, CHUNK)
    dst_p = jnp.concatenate([dst, pad]).reshape(NW * CPW, CHUNK)
    src_f = src_p.reshape(E_PAD)
    dst_f = dst_p.reshape(E_PAD)

    BLK_E = 5120
    NBLK_E = E // BLK_E
    stats = _tc_call(
        _ef_stats_body, (NBLK_E,),
        [pl.BlockSpec((BLK_E, 16), lambda i: (i, 0))],
        pl.BlockSpec((2, 16), lambda i: (0, 0)),
        jax.ShapeDtypeStruct((2, 16), _f32),
    )(edges_features)

    wa1 = jnp.concatenate([p['ad_W1'], p['ar_W1']], axis=1)
    ba1 = jnp.concatenate([p['ad_b1'], p['ar_b1']]).reshape(1, 32)
    wa2 = jnp.zeros((32, 2), _f32)
    wa2 = wa2.at[:16, 0:1].set(p['ad_W2']).at[16:, 1:2].set(p['ar_W2'])
    ba2 = jnp.concatenate([p['ad_b2'], p['ar_b2']]).reshape(1, 2)

    cw = lambda s: pl.BlockSpec(s, lambda i: tuple(0 for _ in s))
    clamp = lambda i: (jnp.minimum(i, NBLK_E - 1), 0)
    efn, ef_nn, a2_p, ad_f, ar_f = _tc_call(
        _edge_mlp_body, (E_PAD // BLK_E,),
        [pl.BlockSpec((BLK_E, 16), clamp), cw((2, 16)),
         cw((16, 16)), cw((1, 16)), cw((16, 16)), cw((1, 16)),
         cw((16, 32)), cw((1, 32)), cw((32, 2)), cw((1, 2))],
        [pl.BlockSpec((BLK_E, 16), lambda i: (i, 0)),
         pl.BlockSpec((BLK_E, 16), lambda i: (i, 0)),
         pl.BlockSpec((BLK_E, 2), lambda i: (i, 0)),
         pl.BlockSpec((BLK_E,), lambda i: (i,)),
         pl.BlockSpec((BLK_E,), lambda i: (i,))],
        [jax.ShapeDtypeStruct((E_PAD, 16), _f32),
         jax.ShapeDtypeStruct((E_PAD, 16), _f32),
         jax.ShapeDtypeStruct((E_PAD, 2), _f32),
         jax.ShapeDtypeStruct((E_PAD,), _f32),
         jax.ShapeDtypeStruct((E_PAD,), _f32)],
        parallel=True,
    )(edges_features, stats, p['ein_W1'], p['ein_b1'].reshape(1, 16),
      p['ein_W2'], p['ein_b2'].reshape(1, 16), wa1, ba1, wa2, ba2)

    node = _tc_call(
        _node_mlp_body, (1,),
        [cw((N, IN)), cw((IN, H)), cw((1, H)), cw((H, H)), cw((1, H))],
        cw((N, H)),
        jax.ShapeDtypeStruct((N, H), _f32),
    )(x, p['inp_W1'], p['inp_b1'].reshape(1, H),
      p['inp_W2'], p['inp_b2'].reshape(1, H))

    dparts = den_scatter(src_f, dst_f, ad_f, ar_f)
    den = _tc_call(
        _den_reduce_body, (1,),
        [cw((NW, N))], cw((N,)),
        jax.ShapeDtypeStruct((N,), _f32),
    )(dparts)
    den_col = den.reshape(N, 1)

    zero_nh = jnp.zeros((N, H), _f32)
    BLK_M = 4096
    for name in ('gc0', 'gc1'):
        w1 = p[name + '_W1']
        w1a, w1b = w1[:H], w1[H:]
        b1 = p[name + '_b1'].reshape(1, 2 * H)
        w2 = p[name + '_W2']
        b2 = p[name + '_b2'].reshape(1, H)

        wd = jnp.concatenate([w1a - w1b, w1b], axis=0)
        wr = jnp.concatenate([w1b, w1a - w1b], axis=0)
        w1p = jnp.concatenate([wd, wr], axis=1)
        b1p = jnp.concatenate([b1, b1], axis=1)
        w2p = jnp.zeros((4 * H, 2 * H), _f32)
        w2p = w2p.at[:2 * H, :H].set(w2).at[2 * H:, H:].set(w2)
        b2p = jnp.concatenate([b2, b2], axis=1)

        hs, hd = gather2(node, src_p, dst_p)
        md, mr = _tc_call(
            _msg_mlp_body, (E_PAD // BLK_M,),
            [pl.BlockSpec((BLK_M, H), lambda i: (i, 0)),
             pl.BlockSpec((BLK_M, H), lambda i: (i, 0)),
             pl.BlockSpec((BLK_M, 2), lambda i: (i, 0)),
             cw((2 * H, 4 * H)), cw((1, 4 * H)),
             cw((4 * H, 2 * H)), cw((1, 2 * H))],
            [pl.BlockSpec((BLK_M, H), lambda i: (i, 0)),
             pl.BlockSpec((BLK_M, H), lambda i: (i, 0))],
            [jax.ShapeDtypeStruct((E_PAD, H), _f32),
             jax.ShapeDtypeStruct((E_PAD, H), _f32)],
            parallel=True,
        )(hs, hd, a2_p, w1p, b1p, w2p, b2p)

        parts = scatter2(md, mr, src_p, dst_p, zero_nh)
        node = _tc_call(
            _node_update_body, (1,),
            [cw((N, H)), cw((NC, N, H)), cw((N, 1)),
             cw((H, 2 * H)), cw((1, 2 * H)), cw((2 * H, H)), cw((1, H))],
            cw((N, H)),
            jax.ShapeDtypeStruct((N, H), _f32),
        )(node, parts, den_col, w1a, b1, w2, b2)

    hs, hd = gather2(node, src_p, dst_p)
    BLK_O = 5120
    pred = _tc_call(
        _edge_out_body, (E // BLK_O,),
        [pl.BlockSpec((BLK_O, H), lambda i: (i, 0)),
         pl.BlockSpec((BLK_O, H), lambda i: (i, 0)),
         pl.BlockSpec((BLK_O, 16), lambda i: (i, 0)),
         pl.BlockSpec((BLK_O, 16), lambda i: (i, 0)),
         cw((2 * H + 32, H)), cw((1, H)), cw((H, 1)), cw((1, 1))],
        pl.BlockSpec((BLK_O,), lambda i: (i,)),
        jax.ShapeDtypeStruct((E,), _f32),
        parallel=True,
    )(hs, hd, ef_nn, efn, p['en_W1'],
      p['en_b1'].reshape(1, H), p['en_W2'], p['en_b2'].reshape(1, 1))
    return pred
